# baseline (device time: 35011 ns/iter reference)
import jax
import jax.numpy as jnp
from jax import lax
from jax.experimental import pallas as pl
from jax.experimental.pallas import tpu as pltpu

N_DEV = 4
B_LOC = 2
SQ = 256
SKV = 256
HQ = 16
H_BLK = 4
DH = 64
D_MODEL = 512
D_BLK = H_BLK * DH


def kernel(x, Wq, K_ext, V_ext, Wo):
    def body(x_ref, wq_ref, k_hbm, v_hbm, wo_ref, out_ref,
             wq_comm, wo_comm, wq_send, wo_send,
             send_sems, recv_sems, copy_sems, k_vmem, v_vmem, acc, ctx_buf):
        my = lax.axis_index("i")

        kd = pltpu.make_async_copy(
            k_hbm.at[pl.ds(B_LOC * my, B_LOC)], k_vmem, copy_sems.at[0])
        vd = pltpu.make_async_copy(
            v_hbm.at[pl.ds(B_LOC * my, B_LOC)], v_vmem, copy_sems.at[1])
        kd.start()
        vd.start()

        wq_send[:] = wq_ref[:].astype(jnp.bfloat16)
        wo_send[:] = wo_ref[:].astype(jnp.bfloat16)

        barrier = pltpu.get_barrier_semaphore()
        for k in range(1, N_DEV):
            pl.semaphore_signal(
                barrier, inc=1,
                device_id=(lax.rem(my + k, N_DEV),),
                device_id_type=pl.DeviceIdType.MESH,
            )
        pl.semaphore_wait(barrier, N_DEV - 1)

        sends = []
        for k in range(1, N_DEV):
            dst = lax.rem(my + k, N_DEV)
            r_wq = pltpu.make_async_remote_copy(
                src_ref=wq_send,
                dst_ref=wq_comm.at[k - 1],
                send_sem=send_sems.at[2 * (k - 1)],
                recv_sem=recv_sems.at[2 * (k - 1)],
                device_id=(dst,),
                device_id_type=pl.DeviceIdType.MESH,
            )
            r_wo = pltpu.make_async_remote_copy(
                src_ref=wo_send,
                dst_ref=wo_comm.at[k - 1],
                send_sem=send_sems.at[2 * (k - 1) + 1],
                recv_sem=recv_sems.at[2 * (k - 1) + 1],
                device_id=(dst,),
                device_id_type=pl.DeviceIdType.MESH,
            )
            r_wq.start()
            r_wo.start()
            sends.append(r_wq)
            sends.append(r_wo)

        qi = lax.broadcasted_iota(jnp.int32, (SQ, SKV), 0)
        ki = lax.broadcasted_iota(jnp.int32, (SQ, SKV), 1)
        mask = (jnp.abs(qi - ki) <= 128) | (ki < 32) | (qi < 32)

        x2d = x_ref[:].reshape(B_LOC * SQ, D_MODEL).astype(jnp.bfloat16)

        kd.wait()
        vd.wait()

        def do_block(origin, wq_blk, wo_blk):
            q = lax.dot_general(
                x2d, wq_blk, (((1,), (0,)), ((), ())),
                preferred_element_type=jnp.float32,
            )
            q_bf = q.astype(jnp.bfloat16)
            for b in range(B_LOC):
                for p in range(H_BLK // 2):
                    off = pl.multiple_of(origin * D_BLK, 128) + p * 128
                    k_pair = k_vmem[b, :, pl.ds(off, 2 * DH)].astype(
                        jnp.bfloat16)
                    v_pair = v_vmem[b, :, pl.ds(off, 2 * DH)].astype(
                        jnp.bfloat16)
                    for hh in range(2):
                        h = 2 * p + hh
                        qh = q_bf[b * SQ:(b + 1) * SQ, h * DH:(h + 1) * DH]
                        kh = k_pair[:, hh * DH:(hh + 1) * DH]
                        vh = v_pair[:, hh * DH:(hh + 1) * DH]
                        sc = lax.dot_general(
                            qh, kh, (((1,), (1,)), ((), ())),
                            preferred_element_type=jnp.float32,
                        ) * 0.125
                        sc = jnp.where(mask, sc, -1e9)
                        m = jnp.max(sc, axis=1, keepdims=True)
                        w = jnp.exp(sc - m)
                        w = w / jnp.sum(w, axis=1, keepdims=True)
                        ctx = lax.dot_general(
                            w.astype(jnp.bfloat16), vh, (((1,), (0,)), ((), ())),
                            preferred_element_type=jnp.float32,
                        )
                        ctx_buf[b * SQ:(b + 1) * SQ, h * DH:(h + 1) * DH] = (
                            ctx.astype(jnp.bfloat16)
                        )
            return lax.dot_general(
                ctx_buf[:], wo_blk, (((1,), (0,)), ((), ())),
                preferred_element_type=jnp.float32,
            )

        acc[:] = do_block(my, wq_send[:], wo_send[:])

        for k in (1, 3, 2):
            recv_wq = pltpu.make_async_remote_copy(
                src_ref=wq_send,
                dst_ref=wq_comm.at[k - 1],
                send_sem=send_sems.at[2 * (k - 1)],
                recv_sem=recv_sems.at[2 * (k - 1)],
                device_id=(my,),
                device_id_type=pl.DeviceIdType.MESH,
            )
            recv_wo = pltpu.make_async_remote_copy(
                src_ref=wo_send,
                dst_ref=wo_comm.at[k - 1],
                send_sem=send_sems.at[2 * (k - 1) + 1],
                recv_sem=recv_sems.at[2 * (k - 1) + 1],
                device_id=(my,),
                device_id_type=pl.DeviceIdType.MESH,
            )
            recv_wq.wait_recv()
            recv_wo.wait_recv()
            origin = lax.rem(my + (N_DEV - k), N_DEV)
            acc[:] += do_block(origin, wq_comm[k - 1], wo_comm[k - 1])

        for r in sends:
            r.wait_send()

        out_ref[:] = acc[:].reshape(B_LOC, SQ, D_MODEL)

    return pl.pallas_call(
        body,
        out_shape=jax.ShapeDtypeStruct((B_LOC, SQ, D_MODEL), jnp.float32),
        in_specs=[
            pl.BlockSpec(memory_space=pltpu.VMEM),
            pl.BlockSpec(memory_space=pltpu.VMEM),
            pl.BlockSpec(memory_space=pl.ANY),
            pl.BlockSpec(memory_space=pl.ANY),
            pl.BlockSpec(memory_space=pltpu.VMEM),
        ],
        out_specs=pl.BlockSpec(memory_space=pltpu.VMEM),
        scratch_shapes=[
            pltpu.VMEM((N_DEV - 1, D_MODEL, D_BLK), jnp.bfloat16),
            pltpu.VMEM((N_DEV - 1, D_BLK, D_MODEL), jnp.bfloat16),
            pltpu.VMEM((D_MODEL, D_BLK), jnp.bfloat16),
            pltpu.VMEM((D_BLK, D_MODEL), jnp.bfloat16),
            pltpu.SemaphoreType.DMA((2 * (N_DEV - 1),)),
            pltpu.SemaphoreType.DMA((2 * (N_DEV - 1),)),
            pltpu.SemaphoreType.DMA((2,)),
            pltpu.VMEM((B_LOC, SKV, HQ * DH), jnp.float32),
            pltpu.VMEM((B_LOC, SKV, HQ * DH), jnp.float32),
            pltpu.VMEM((B_LOC * SQ, D_MODEL), jnp.float32),
            pltpu.VMEM((B_LOC * SQ, D_BLK), jnp.bfloat16),
        ],
        compiler_params=pltpu.CompilerParams(collective_id=0),
    )(x, Wq,
      K_ext.reshape(K_ext.shape[0], SKV, HQ * DH),
      V_ext.reshape(V_ext.shape[0], SKV, HQ * DH),
      Wo)


# device time: 25882 ns/iter; 1.3527x vs baseline; 1.3527x over previous
import jax
import jax.numpy as jnp
from jax import lax
from jax.experimental import pallas as pl
from jax.experimental.pallas import tpu as pltpu

N_DEV = 4
B_LOC = 2
SQ = 256
SKV = 256
HQ = 16
H_BLK = 4
DH = 64
D_MODEL = 512
D_BLK = H_BLK * DH


def kernel(x, Wq, K_ext, V_ext, Wo):
    i = lax.axis_index("i")
    Kc = lax.dynamic_slice_in_dim(K_ext, B_LOC * i, B_LOC, axis=0).astype(
        jnp.bfloat16).reshape(B_LOC, SKV, HQ * DH)
    Vc = lax.dynamic_slice_in_dim(V_ext, B_LOC * i, B_LOC, axis=0).astype(
        jnp.bfloat16).reshape(B_LOC, SKV, HQ * DH)

    def body(x_ref, wq_ref, k_ref, v_ref, wo_ref, out_ref,
             wq_comm, wo_comm, wq_send, wo_send,
             send_sems, recv_sems, fwd_send_sems, fwd_recv_sems,
             acc, ctx_buf):
        my = lax.axis_index("i")
        right = lax.rem(my + 1, N_DEV)
        left = lax.rem(my + N_DEV - 1, N_DEV)

        wq_send[:] = wq_ref[:].astype(jnp.bfloat16)
        wo_send[:] = wo_ref[:].astype(jnp.bfloat16)

        barrier = pltpu.get_barrier_semaphore()
        for k in range(1, N_DEV):
            pl.semaphore_signal(
                barrier, inc=1,
                device_id=(lax.rem(my + k, N_DEV),),
                device_id_type=pl.DeviceIdType.MESH,
            )
        pl.semaphore_wait(barrier, N_DEV - 1)

        sends = []
        for k, slot in ((1, 0), (3, 2)):
            dst = lax.rem(my + k, N_DEV)
            r_wq = pltpu.make_async_remote_copy(
                src_ref=wq_send,
                dst_ref=wq_comm.at[slot],
                send_sem=send_sems.at[2 * slot],
                recv_sem=recv_sems.at[2 * slot],
                device_id=(dst,),
                device_id_type=pl.DeviceIdType.MESH,
            )
            r_wo = pltpu.make_async_remote_copy(
                src_ref=wo_send,
                dst_ref=wo_comm.at[slot],
                send_sem=send_sems.at[2 * slot + 1],
                recv_sem=recv_sems.at[2 * slot + 1],
                device_id=(dst,),
                device_id_type=pl.DeviceIdType.MESH,
            )
            r_wq.start()
            r_wo.start()
            sends.append(r_wq)
            sends.append(r_wo)

        qi = lax.broadcasted_iota(jnp.int32, (SQ, SKV), 0)
        ki = lax.broadcasted_iota(jnp.int32, (SQ, SKV), 1)
        mask = (jnp.abs(qi - ki) <= 128) | (ki < 32) | (qi < 32)

        x2d = x_ref[:].reshape(B_LOC * SQ, D_MODEL).astype(jnp.bfloat16)

        def do_block(origin, wq_blk, wo_blk):
            q = lax.dot_general(
                x2d, wq_blk, (((1,), (0,)), ((), ())),
                preferred_element_type=jnp.float32,
            )
            q_bf = q.astype(jnp.bfloat16)
            for b in range(B_LOC):
                for p in range(H_BLK // 2):
                    off = pl.multiple_of(origin * D_BLK, 128) + p * 128
                    k_pair = k_ref[b, :, pl.ds(off, 2 * DH)]
                    v_pair = v_ref[b, :, pl.ds(off, 2 * DH)]
                    for hh in range(2):
                        h = 2 * p + hh
                        qh = q_bf[b * SQ:(b + 1) * SQ, h * DH:(h + 1) * DH]
                        kh = k_pair[:, hh * DH:(hh + 1) * DH]
                        vh = v_pair[:, hh * DH:(hh + 1) * DH]
                        sc = lax.dot_general(
                            qh, kh, (((1,), (1,)), ((), ())),
                            preferred_element_type=jnp.float32,
                        ) * 0.125
                        sc = jnp.where(mask, sc, -1e9)
                        m = jnp.max(sc, axis=1, keepdims=True)
                        w = jnp.exp(sc - m)
                        w = w / jnp.sum(w, axis=1, keepdims=True)
                        ctx = lax.dot_general(
                            w.astype(jnp.bfloat16), vh, (((1,), (0,)), ((), ())),
                            preferred_element_type=jnp.float32,
                        )
                        ctx_buf[b * SQ:(b + 1) * SQ, h * DH:(h + 1) * DH] = (
                            ctx.astype(jnp.bfloat16)
                        )
            return lax.dot_general(
                ctx_buf[:], wo_blk, (((1,), (0,)), ((), ())),
                preferred_element_type=jnp.float32,
            )

        def recv_pair(slot):
            rq = pltpu.make_async_remote_copy(
                src_ref=wq_send,
                dst_ref=wq_comm.at[slot],
                send_sem=send_sems.at[2 * slot],
                recv_sem=recv_sems.at[2 * slot],
                device_id=(my,),
                device_id_type=pl.DeviceIdType.MESH,
            )
            ro = pltpu.make_async_remote_copy(
                src_ref=wo_send,
                dst_ref=wo_comm.at[slot],
                send_sem=send_sems.at[2 * slot + 1],
                recv_sem=recv_sems.at[2 * slot + 1],
                device_id=(my,),
                device_id_type=pl.DeviceIdType.MESH,
            )
            rq.wait_recv()
            ro.wait_recv()

        acc[:] = do_block(my, wq_send[:], wo_send[:])

        WQ_H = D_MODEL // 2
        WO_H = D_BLK // 2

        recv_pair(0)
        f0q = pltpu.make_async_remote_copy(
            src_ref=wq_comm.at[0, pl.ds(0, WQ_H)],
            dst_ref=wq_comm.at[1, pl.ds(0, WQ_H)],
            send_sem=fwd_send_sems.at[0],
            recv_sem=fwd_recv_sems.at[0],
            device_id=(right,),
            device_id_type=pl.DeviceIdType.MESH,
        )
        f0o = pltpu.make_async_remote_copy(
            src_ref=wo_comm.at[0, pl.ds(0, WO_H)],
            dst_ref=wo_comm.at[1, pl.ds(0, WO_H)],
            send_sem=fwd_send_sems.at[1],
            recv_sem=fwd_recv_sems.at[1],
            device_id=(right,),
            device_id_type=pl.DeviceIdType.MESH,
        )
        f0q.start()
        f0o.start()
        acc[:] += do_block(left, wq_comm[0], wo_comm[0])

        recv_pair(2)
        f2q = pltpu.make_async_remote_copy(
            src_ref=wq_comm.at[2, pl.ds(WQ_H, WQ_H)],
            dst_ref=wq_comm.at[1, pl.ds(WQ_H, WQ_H)],
            send_sem=fwd_send_sems.at[2],
            recv_sem=fwd_recv_sems.at[2],
            device_id=(left,),
            device_id_type=pl.DeviceIdType.MESH,
        )
        f2o = pltpu.make_async_remote_copy(
            src_ref=wo_comm.at[2, pl.ds(WO_H, WO_H)],
            dst_ref=wo_comm.at[1, pl.ds(WO_H, WO_H)],
            send_sem=fwd_send_sems.at[3],
            recv_sem=fwd_recv_sems.at[3],
            device_id=(left,),
            device_id_type=pl.DeviceIdType.MESH,
        )
        f2q.start()
        f2o.start()
        acc[:] += do_block(right, wq_comm[2], wo_comm[2])

        for idx in range(4):
            srcs = (wq_comm.at[0, pl.ds(0, WQ_H)],
                    wo_comm.at[0, pl.ds(0, WO_H)],
                    wq_comm.at[2, pl.ds(WQ_H, WQ_H)],
                    wo_comm.at[2, pl.ds(WO_H, WO_H)])
            dsts = (wq_comm.at[1, pl.ds(0, WQ_H)],
                    wo_comm.at[1, pl.ds(0, WO_H)],
                    wq_comm.at[1, pl.ds(WQ_H, WQ_H)],
                    wo_comm.at[1, pl.ds(WO_H, WO_H)])
            fr = pltpu.make_async_remote_copy(
                src_ref=srcs[idx],
                dst_ref=dsts[idx],
                send_sem=fwd_send_sems.at[idx],
                recv_sem=fwd_recv_sems.at[idx],
                device_id=(my,),
                device_id_type=pl.DeviceIdType.MESH,
            )
            fr.wait_recv()
        diag = lax.rem(my + 2, N_DEV)
        acc[:] += do_block(diag, wq_comm[1], wo_comm[1])

        for r in sends:
            r.wait_send()
        for f in (f0q, f0o, f2q, f2o):
            f.wait_send()

        out_ref[:] = acc[:].reshape(B_LOC, SQ, D_MODEL)

    return pl.pallas_call(
        body,
        out_shape=jax.ShapeDtypeStruct((B_LOC, SQ, D_MODEL), jnp.float32),
        in_specs=[
            pl.BlockSpec(memory_space=pltpu.VMEM),
            pl.BlockSpec(memory_space=pltpu.VMEM),
            pl.BlockSpec(memory_space=pltpu.VMEM),
            pl.BlockSpec(memory_space=pltpu.VMEM),
            pl.BlockSpec(memory_space=pltpu.VMEM),
        ],
        out_specs=pl.BlockSpec(memory_space=pltpu.VMEM),
        scratch_shapes=[
            pltpu.VMEM((N_DEV - 1, D_MODEL, D_BLK), jnp.bfloat16),
            pltpu.VMEM((N_DEV - 1, D_BLK, D_MODEL), jnp.bfloat16),
            pltpu.VMEM((D_MODEL, D_BLK), jnp.bfloat16),
            pltpu.VMEM((D_BLK, D_MODEL), jnp.bfloat16),
            pltpu.SemaphoreType.DMA((6,)),
            pltpu.SemaphoreType.DMA((6,)),
            pltpu.SemaphoreType.DMA((4,)),
            pltpu.SemaphoreType.DMA((4,)),
            pltpu.VMEM((B_LOC * SQ, D_MODEL), jnp.float32),
            pltpu.VMEM((B_LOC * SQ, D_BLK), jnp.bfloat16),
        ],
        compiler_params=pltpu.CompilerParams(collective_id=0),
    )(x, Wq, Kc, Vc, Wo)


# device time: 22272 ns/iter; 1.5720x vs baseline; 1.1621x over previous
import jax
import jax.numpy as jnp
from jax import lax
from jax.experimental import pallas as pl
from jax.experimental.pallas import tpu as pltpu

N_DEV = 4
B_LOC = 2
SQ = 256
SKV = 256
HQ = 16
H_BLK = 4
DH = 64
D_MODEL = 512
D_BLK = H_BLK * DH


def kernel(x, Wq, K_ext, V_ext, Wo):
    i = lax.axis_index("i")
    Kc = lax.dynamic_slice_in_dim(K_ext, B_LOC * i, B_LOC, axis=0).astype(
        jnp.bfloat16).reshape(B_LOC, SKV, HQ * DH)
    Vc = lax.dynamic_slice_in_dim(V_ext, B_LOC * i, B_LOC, axis=0).astype(
        jnp.bfloat16).reshape(B_LOC, SKV, HQ * DH)

    def body(x_ref, wq_ref, k_ref, v_ref, wo_ref, out_ref,
             wq_comm, wo_comm, wq_send, wo_send,
             send_sems, recv_sems, fwd_send_sems, fwd_recv_sems,
             acc, ctx_buf):
        my = lax.axis_index("i")
        right = lax.rem(my + 1, N_DEV)
        left = lax.rem(my + N_DEV - 1, N_DEV)

        wq_send[:] = wq_ref[:].astype(jnp.bfloat16)
        wo_send[:] = wo_ref[:].astype(jnp.bfloat16)

        barrier = pltpu.get_barrier_semaphore()
        for k in range(1, N_DEV):
            pl.semaphore_signal(
                barrier, inc=1,
                device_id=(lax.rem(my + k, N_DEV),),
                device_id_type=pl.DeviceIdType.MESH,
            )
        pl.semaphore_wait(barrier, N_DEV - 1)

        sends = []
        for k, slot in ((1, 0), (3, 2)):
            dst = lax.rem(my + k, N_DEV)
            r_wq = pltpu.make_async_remote_copy(
                src_ref=wq_send,
                dst_ref=wq_comm.at[slot],
                send_sem=send_sems.at[2 * slot],
                recv_sem=recv_sems.at[2 * slot],
                device_id=(dst,),
                device_id_type=pl.DeviceIdType.MESH,
            )
            r_wo = pltpu.make_async_remote_copy(
                src_ref=wo_send,
                dst_ref=wo_comm.at[slot],
                send_sem=send_sems.at[2 * slot + 1],
                recv_sem=recv_sems.at[2 * slot + 1],
                device_id=(dst,),
                device_id_type=pl.DeviceIdType.MESH,
            )
            r_wq.start()
            r_wo.start()
            sends.append(r_wq)
            sends.append(r_wo)

        qi = lax.broadcasted_iota(jnp.int32, (SQ, SKV), 0)
        ki = lax.broadcasted_iota(jnp.int32, (SQ, SKV), 1)
        mask = (jnp.abs(qi - ki) <= 128) | (ki < 32) | (qi < 32)

        x2d = x_ref[:].reshape(B_LOC * SQ, D_MODEL).astype(jnp.bfloat16)

        def do_block(origin, wq_blk, wo_blk):
            q = lax.dot_general(
                x2d, wq_blk, (((1,), (0,)), ((), ())),
                preferred_element_type=jnp.float32,
            )
            q_bf = q.astype(jnp.bfloat16)
            for b in range(B_LOC):
                for p in range(H_BLK // 2):
                    off = pl.multiple_of(origin * D_BLK, 128) + p * 128
                    k_pair = k_ref[b, :, pl.ds(off, 2 * DH)]
                    v_pair = v_ref[b, :, pl.ds(off, 2 * DH)]
                    for hh in range(2):
                        h = 2 * p + hh
                        qh = q_bf[b * SQ:(b + 1) * SQ, h * DH:(h + 1) * DH]
                        kh = k_pair[:, hh * DH:(hh + 1) * DH]
                        vh = v_pair[:, hh * DH:(hh + 1) * DH]
                        sc = lax.dot_general(
                            qh, kh, (((1,), (1,)), ((), ())),
                            preferred_element_type=jnp.float32,
                        ) * 0.125
                        sc = jnp.where(mask, sc, -1e9)
                        w = jnp.exp(sc)
                        denom = jnp.sum(w, axis=1, keepdims=True)
                        ctx = lax.dot_general(
                            w.astype(jnp.bfloat16), vh, (((1,), (0,)), ((), ())),
                            preferred_element_type=jnp.float32,
                        ) / denom
                        ctx_buf[b * SQ:(b + 1) * SQ, h * DH:(h + 1) * DH] = (
                            ctx.astype(jnp.bfloat16)
                        )
            return lax.dot_general(
                ctx_buf[:], wo_blk, (((1,), (0,)), ((), ())),
                preferred_element_type=jnp.float32,
            )

        def recv_pair(slot):
            rq = pltpu.make_async_remote_copy(
                src_ref=wq_send,
                dst_ref=wq_comm.at[slot],
                send_sem=send_sems.at[2 * slot],
                recv_sem=recv_sems.at[2 * slot],
                device_id=(my,),
                device_id_type=pl.DeviceIdType.MESH,
            )
            ro = pltpu.make_async_remote_copy(
                src_ref=wo_send,
                dst_ref=wo_comm.at[slot],
                send_sem=send_sems.at[2 * slot + 1],
                recv_sem=recv_sems.at[2 * slot + 1],
                device_id=(my,),
                device_id_type=pl.DeviceIdType.MESH,
            )
            rq.wait_recv()
            ro.wait_recv()

        acc[:] = do_block(my, wq_send[:], wo_send[:])

        WQ_H = D_MODEL // 2
        WO_H = D_BLK // 2

        recv_pair(0)
        f0q = pltpu.make_async_remote_copy(
            src_ref=wq_comm.at[0, pl.ds(0, WQ_H)],
            dst_ref=wq_comm.at[1, pl.ds(0, WQ_H)],
            send_sem=fwd_send_sems.at[0],
            recv_sem=fwd_recv_sems.at[0],
            device_id=(right,),
            device_id_type=pl.DeviceIdType.MESH,
        )
        f0o = pltpu.make_async_remote_copy(
            src_ref=wo_comm.at[0, pl.ds(0, WO_H)],
            dst_ref=wo_comm.at[1, pl.ds(0, WO_H)],
            send_sem=fwd_send_sems.at[1],
            recv_sem=fwd_recv_sems.at[1],
            device_id=(right,),
            device_id_type=pl.DeviceIdType.MESH,
        )
        f0q.start()
        f0o.start()
        recv_pair(2)
        f2q = pltpu.make_async_remote_copy(
            src_ref=wq_comm.at[2, pl.ds(WQ_H, WQ_H)],
            dst_ref=wq_comm.at[1, pl.ds(WQ_H, WQ_H)],
            send_sem=fwd_send_sems.at[2],
            recv_sem=fwd_recv_sems.at[2],
            device_id=(left,),
            device_id_type=pl.DeviceIdType.MESH,
        )
        f2o = pltpu.make_async_remote_copy(
            src_ref=wo_comm.at[2, pl.ds(WO_H, WO_H)],
            dst_ref=wo_comm.at[1, pl.ds(WO_H, WO_H)],
            send_sem=fwd_send_sems.at[3],
            recv_sem=fwd_recv_sems.at[3],
            device_id=(left,),
            device_id_type=pl.DeviceIdType.MESH,
        )
        f2q.start()
        f2o.start()

        acc[:] += do_block(left, wq_comm[0], wo_comm[0])
        acc[:] += do_block(right, wq_comm[2], wo_comm[2])

        for idx in range(4):
            srcs = (wq_comm.at[0, pl.ds(0, WQ_H)],
                    wo_comm.at[0, pl.ds(0, WO_H)],
                    wq_comm.at[2, pl.ds(WQ_H, WQ_H)],
                    wo_comm.at[2, pl.ds(WO_H, WO_H)])
            dsts = (wq_comm.at[1, pl.ds(0, WQ_H)],
                    wo_comm.at[1, pl.ds(0, WO_H)],
                    wq_comm.at[1, pl.ds(WQ_H, WQ_H)],
                    wo_comm.at[1, pl.ds(WO_H, WO_H)])
            fr = pltpu.make_async_remote_copy(
                src_ref=srcs[idx],
                dst_ref=dsts[idx],
                send_sem=fwd_send_sems.at[idx],
                recv_sem=fwd_recv_sems.at[idx],
                device_id=(my,),
                device_id_type=pl.DeviceIdType.MESH,
            )
            fr.wait_recv()
        diag = lax.rem(my + 2, N_DEV)
        acc[:] += do_block(diag, wq_comm[1], wo_comm[1])

        for r in sends:
            r.wait_send()
        for f in (f0q, f0o, f2q, f2o):
            f.wait_send()

        out_ref[:] = acc[:].reshape(B_LOC, SQ, D_MODEL)

    return pl.pallas_call(
        body,
        out_shape=jax.ShapeDtypeStruct((B_LOC, SQ, D_MODEL), jnp.float32),
        in_specs=[
            pl.BlockSpec(memory_space=pltpu.VMEM),
            pl.BlockSpec(memory_space=pltpu.VMEM),
            pl.BlockSpec(memory_space=pltpu.VMEM),
            pl.BlockSpec(memory_space=pltpu.VMEM),
            pl.BlockSpec(memory_space=pltpu.VMEM),
        ],
        out_specs=pl.BlockSpec(memory_space=pltpu.VMEM),
        scratch_shapes=[
            pltpu.VMEM((N_DEV - 1, D_MODEL, D_BLK), jnp.bfloat16),
            pltpu.VMEM((N_DEV - 1, D_BLK, D_MODEL), jnp.bfloat16),
            pltpu.VMEM((D_MODEL, D_BLK), jnp.bfloat16),
            pltpu.VMEM((D_BLK, D_MODEL), jnp.bfloat16),
            pltpu.SemaphoreType.DMA((6,)),
            pltpu.SemaphoreType.DMA((6,)),
            pltpu.SemaphoreType.DMA((4,)),
            pltpu.SemaphoreType.DMA((4,)),
            pltpu.VMEM((B_LOC * SQ, D_MODEL), jnp.float32),
            pltpu.VMEM((B_LOC * SQ, D_BLK), jnp.bfloat16),
        ],
        compiler_params=pltpu.CompilerParams(collective_id=0),
    )(x, Wq, Kc, Vc, Wo)


# device time: 21188 ns/iter; 1.6524x vs baseline; 1.0512x over previous
import jax
import jax.numpy as jnp
from jax import lax
from jax.experimental import pallas as pl
from jax.experimental.pallas import tpu as pltpu

N_DEV = 4
B_LOC = 2
SQ = 256
SKV = 256
HQ = 16
H_BLK = 4
DH = 64
D_MODEL = 512
D_BLK = H_BLK * DH


def kernel(x, Wq, K_ext, V_ext, Wo):
    i = lax.axis_index("i")
    Kc = lax.dynamic_slice_in_dim(K_ext, B_LOC * i, B_LOC, axis=0).astype(
        jnp.bfloat16).reshape(B_LOC, SKV, HQ * DH)
    Vc = lax.dynamic_slice_in_dim(V_ext, B_LOC * i, B_LOC, axis=0).astype(
        jnp.bfloat16).reshape(B_LOC, SKV, HQ * DH)

    def body(x_ref, wq_ref, k_ref, v_ref, wo_ref, out_ref,
             wq_comm, wo_comm, wq_send, wo_send,
             send_sems, recv_sems, fwd_send_sems, fwd_recv_sems,
             acc, ctx_buf):
        my = lax.axis_index("i")
        right = lax.rem(my + 1, N_DEV)
        left = lax.rem(my + N_DEV - 1, N_DEV)

        wq_send[:] = wq_ref[:].astype(jnp.bfloat16)
        wo_send[:] = wo_ref[:].astype(jnp.bfloat16)

        barrier = pltpu.get_barrier_semaphore()
        for k in range(1, N_DEV):
            pl.semaphore_signal(
                barrier, inc=1,
                device_id=(lax.rem(my + k, N_DEV),),
                device_id_type=pl.DeviceIdType.MESH,
            )
        pl.semaphore_wait(barrier, N_DEV - 1)

        sends = []
        for k, slot in ((1, 0), (3, 2)):
            dst = lax.rem(my + k, N_DEV)
            r_wq = pltpu.make_async_remote_copy(
                src_ref=wq_send,
                dst_ref=wq_comm.at[slot],
                send_sem=send_sems.at[2 * slot],
                recv_sem=recv_sems.at[2 * slot],
                device_id=(dst,),
                device_id_type=pl.DeviceIdType.MESH,
            )
            r_wo = pltpu.make_async_remote_copy(
                src_ref=wo_send,
                dst_ref=wo_comm.at[slot],
                send_sem=send_sems.at[2 * slot + 1],
                recv_sem=recv_sems.at[2 * slot + 1],
                device_id=(dst,),
                device_id_type=pl.DeviceIdType.MESH,
            )
            r_wq.start()
            r_wo.start()
            sends.append(r_wq)
            sends.append(r_wo)

        qi = lax.broadcasted_iota(jnp.int32, (SQ, SKV), 0)
        ki = lax.broadcasted_iota(jnp.int32, (SQ, SKV), 1)
        mask = (jnp.abs(qi - ki) <= 128) | (ki < 32) | (qi < 32)

        x2d = x_ref[:].reshape(B_LOC * SQ, D_MODEL).astype(jnp.bfloat16)

        def attn_block(origin, wq_blk):
            q = lax.dot_general(
                x2d, wq_blk, (((1,), (0,)), ((), ())),
                preferred_element_type=jnp.float32,
            )
            q_bf = q.astype(jnp.bfloat16)
            for b in range(B_LOC):
                for p in range(H_BLK // 2):
                    off = pl.multiple_of(origin * D_BLK, 128) + p * 128
                    k_pair = k_ref[b, :, pl.ds(off, 2 * DH)]
                    v_pair = v_ref[b, :, pl.ds(off, 2 * DH)]
                    for hh in range(2):
                        h = 2 * p + hh
                        qh = q_bf[b * SQ:(b + 1) * SQ, h * DH:(h + 1) * DH]
                        kh = k_pair[:, hh * DH:(hh + 1) * DH]
                        vh = v_pair[:, hh * DH:(hh + 1) * DH]
                        sc = lax.dot_general(
                            qh, kh, (((1,), (1,)), ((), ())),
                            preferred_element_type=jnp.float32,
                        ) * 0.125
                        sc = jnp.where(mask, sc, -1e9)
                        w = jnp.exp(sc)
                        denom = jnp.sum(w, axis=1, keepdims=True)
                        ctx = lax.dot_general(
                            w.astype(jnp.bfloat16), vh, (((1,), (0,)), ((), ())),
                            preferred_element_type=jnp.float32,
                        ) / denom
                        ctx_buf[b * SQ:(b + 1) * SQ, h * DH:(h + 1) * DH] = (
                            ctx.astype(jnp.bfloat16)
                        )

        def out_dot(wo_blk):
            return lax.dot_general(
                ctx_buf[:], wo_blk, (((1,), (0,)), ((), ())),
                preferred_element_type=jnp.float32,
            )

        def recv_wq(slot):
            pltpu.make_async_remote_copy(
                src_ref=wq_send,
                dst_ref=wq_comm.at[slot],
                send_sem=send_sems.at[2 * slot],
                recv_sem=recv_sems.at[2 * slot],
                device_id=(my,),
                device_id_type=pl.DeviceIdType.MESH,
            ).wait_recv()

        def recv_wo(slot):
            pltpu.make_async_remote_copy(
                src_ref=wo_send,
                dst_ref=wo_comm.at[slot],
                send_sem=send_sems.at[2 * slot + 1],
                recv_sem=recv_sems.at[2 * slot + 1],
                device_id=(my,),
                device_id_type=pl.DeviceIdType.MESH,
            ).wait_recv()

        WQ_H = D_MODEL // 2
        WO_H = D_BLK // 2
        fwd_srcs = (wq_comm.at[0, pl.ds(0, WQ_H)],
                    wo_comm.at[0, pl.ds(0, WO_H)],
                    wq_comm.at[2, pl.ds(WQ_H, WQ_H)],
                    wo_comm.at[2, pl.ds(WO_H, WO_H)])
        fwd_dsts = (wq_comm.at[1, pl.ds(0, WQ_H)],
                    wo_comm.at[1, pl.ds(0, WO_H)],
                    wq_comm.at[1, pl.ds(WQ_H, WQ_H)],
                    wo_comm.at[1, pl.ds(WO_H, WO_H)])

        def fwd(idx, dst_dev):
            r = pltpu.make_async_remote_copy(
                src_ref=fwd_srcs[idx],
                dst_ref=fwd_dsts[idx],
                send_sem=fwd_send_sems.at[idx],
                recv_sem=fwd_recv_sems.at[idx],
                device_id=(dst_dev,),
                device_id_type=pl.DeviceIdType.MESH,
            )
            return r

        attn_block(my, wq_send[:])
        acc[:] = out_dot(wo_send[:])

        recv_wq(0)
        f0q = fwd(0, right)
        f0q.start()
        attn_block(left, wq_comm[0])
        recv_wo(0)
        f0o = fwd(1, right)
        f0o.start()
        acc[:] += out_dot(wo_comm[0])

        recv_wq(2)
        f2q = fwd(2, left)
        f2q.start()
        attn_block(right, wq_comm[2])
        recv_wo(2)
        f2o = fwd(3, left)
        f2o.start()
        acc[:] += out_dot(wo_comm[2])

        diag = lax.rem(my + 2, N_DEV)
        for idx in (0, 2):
            fwd(idx, my).wait_recv()
        attn_block(diag, wq_comm[1])
        for idx in (1, 3):
            fwd(idx, my).wait_recv()
        acc[:] += out_dot(wo_comm[1])

        for r in sends:
            r.wait_send()
        for f in (f0q, f0o, f2q, f2o):
            f.wait_send()

        out_ref[:] = acc[:].reshape(B_LOC, SQ, D_MODEL)

    return pl.pallas_call(
        body,
        out_shape=jax.ShapeDtypeStruct((B_LOC, SQ, D_MODEL), jnp.float32),
        in_specs=[
            pl.BlockSpec(memory_space=pltpu.VMEM),
            pl.BlockSpec(memory_space=pltpu.VMEM),
            pl.BlockSpec(memory_space=pltpu.VMEM),
            pl.BlockSpec(memory_space=pltpu.VMEM),
            pl.BlockSpec(memory_space=pltpu.VMEM),
        ],
        out_specs=pl.BlockSpec(memory_space=pltpu.VMEM),
        scratch_shapes=[
            pltpu.VMEM((N_DEV - 1, D_MODEL, D_BLK), jnp.bfloat16),
            pltpu.VMEM((N_DEV - 1, D_BLK, D_MODEL), jnp.bfloat16),
            pltpu.VMEM((D_MODEL, D_BLK), jnp.bfloat16),
            pltpu.VMEM((D_BLK, D_MODEL), jnp.bfloat16),
            pltpu.SemaphoreType.DMA((6,)),
            pltpu.SemaphoreType.DMA((6,)),
            pltpu.SemaphoreType.DMA((4,)),
            pltpu.SemaphoreType.DMA((4,)),
            pltpu.VMEM((B_LOC * SQ, D_MODEL), jnp.float32),
            pltpu.VMEM((B_LOC * SQ, D_BLK), jnp.bfloat16),
        ],
        compiler_params=pltpu.CompilerParams(collective_id=0),
    )(x, Wq, Kc, Vc, Wo)
